# baseline (speedup 1.0000x reference)
import functools

import jax
import jax.numpy as jnp
from jax import lax
from jax.experimental import pallas as pl
from jax.experimental.pallas import tpu as pltpu
from jax.experimental.pallas import tpu_sc as plsc

N = 10000
E = 320000
D = 128
H = 128
O = 128
G = 64

NC = 2
NS = 16
NW = NC * NS
CH = 128
EPT = -(-E // NW)
K = -(-EPT // CH)
KF = 79
KS = 2 * K - KF
TOTC = NS * (KF + KS)
EPT_PAD = K * CH
NPAD = 10240
RPT = NPAD // NS
NPAD_S = 10112
RPT_S = NPAD_S // NS
BLK = 1000
NBLK = N // BLK

_sc_mesh = plsc.VectorSubcoreMesh(
    core_axis_name="c", subcore_axis_name="s", num_cores=NC, num_subcores=NS)


@functools.partial(
    pl.kernel,
    out_type=jax.ShapeDtypeStruct((NC, 1, NPAD), jnp.float32),
    mesh=_sc_mesh,
    scratch_types=[
        pltpu.VMEM((max(KF, KS), 2, CH), jnp.int32),
        pltpu.VMEM((CH,), jnp.float32),
        pltpu.VMEM_SHARED((NPAD,), jnp.float32),
    ],
)
def _sc_degree(idx_hbm, ones_hbm, zeros_hbm, out_hbm, idx_d, ones_v, deg_sh):
    cid = lax.axis_index("c")
    sid = lax.axis_index("s")
    base = jnp.where(cid == 0, sid * KF, NS * KF + sid * KS)
    kmine = jnp.where(cid == 0, KF, KS)
    r0 = sid * RPT
    pltpu.sync_copy(zeros_hbm.at[pl.ds(r0, RPT)], deg_sh.at[pl.ds(r0, RPT)])
    pltpu.sync_copy(idx_hbm.at[pl.ds(base, max(KF, KS))], idx_d)
    pltpu.sync_copy(ones_hbm, ones_v)
    plsc.subcore_barrier()

    @pl.loop(0, kmine)
    def _chunks(c):
        pltpu.sync_copy(ones_v, deg_sh.at[idx_d.at[c, 1]], add=True)

    plsc.subcore_barrier()
    pltpu.sync_copy(deg_sh.at[pl.ds(r0, RPT)],
                    out_hbm.at[cid, 0, pl.ds(r0, RPT)])


@functools.partial(
    pl.kernel,
    out_type=jax.ShapeDtypeStruct((NC, NPAD_S, D), jnp.float32),
    mesh=_sc_mesh,
    scratch_types=[
        pltpu.VMEM((4, 2, CH), jnp.int32),
        pltpu.VMEM((3, CH, D), jnp.float32),
        pltpu.VMEM_SHARED((NPAD_S, D), jnp.float32),
        pltpu.SemaphoreType.DMA,
        pltpu.SemaphoreType.DMA,
    ],
)
def _sc_scatter(g_hbm, idx_hbm, zeros_hbm, out_hbm,
                idxb, rows, acc_sh, rsem, isem):
    cid = lax.axis_index("c")
    sid = lax.axis_index("s")
    base = jnp.where(cid == 0, sid * KF, NS * KF + sid * KS)
    kmine = jnp.where(cid == 0, KF, KS)
    r0 = sid * RPT_S
    pltpu.sync_copy(zeros_hbm.at[pl.ds(r0, RPT_S)], acc_sh.at[pl.ds(r0, RPT_S)])
    @pl.when(kmine > 0)
    def _prime():
        pltpu.sync_copy(idx_hbm.at[base], idxb.at[0])
        pltpu.async_copy(g_hbm.at[idxb.at[0, 0]], rows.at[0], rsem)
        pltpu.async_copy(idx_hbm.at[base + 1], idxb.at[1], isem)
        pltpu.async_copy(idx_hbm.at[base + 2], idxb.at[2], isem)
        pltpu.async_copy(idx_hbm.at[base + 3], idxb.at[3], isem)
        pltpu.make_async_copy(idx_hbm.at[0], idxb.at[1], isem).wait()
        pltpu.async_copy(g_hbm.at[idxb.at[1, 0]], rows.at[1], rsem)

    plsc.subcore_barrier()

    @pl.loop(0, kmine)
    def _chunks(c):
        buf = lax.rem(c, 3)
        ibuf = lax.rem(c, 4)
        pltpu.make_async_copy(
            zeros_hbm.at[pl.ds(0, CH)], rows.at[buf], rsem).wait()

        @pl.when(c + 2 < kmine)
        def _next_gather():
            i2 = lax.rem(c + 2, 4)
            pltpu.make_async_copy(idx_hbm.at[0], idxb.at[i2], isem).wait()
            pltpu.async_copy(
                g_hbm.at[idxb.at[i2, 0]], rows.at[lax.rem(c + 2, 3)], rsem)

        pltpu.sync_copy(rows.at[buf], acc_sh.at[idxb.at[ibuf, 1]], add=True)

        @pl.when(c + 4 < kmine)
        def _next_idx():
            pltpu.async_copy(idx_hbm.at[base + c + 4], idxb.at[ibuf], isem)

    plsc.subcore_barrier()
    pltpu.sync_copy(acc_sh.at[pl.ds(r0, RPT_S)],
                    out_hbm.at[cid, pl.ds(r0, RPT_S)])


def _tc0_body(x_ref, w1_ref, m1_ref):
    m1_ref[...] = jnp.dot(x_ref[...], w1_ref[...],
                          preferred_element_type=jnp.float32)


def _tc1_body(m1_ref, degt_ref, g1_ref, dinv_ref):
    deg = degt_ref[:, 0:1] + degt_ref[:, 1:2] + 1.0
    dinv = lax.rsqrt(deg)
    dinv_ref[...] = dinv
    g1_ref[...] = m1_ref[...] * dinv


def _tc2_body(a0_ref, a1_ref, g_ref, dinv_ref, b_ref, w_ref, out_ref):
    dinv = dinv_ref[...]
    h = jax.nn.relu((a0_ref[...] + a1_ref[...] + g_ref[...]) * dinv
                    + b_ref[...])
    out_ref[...] = jnp.dot(h, w_ref[...],
                           preferred_element_type=jnp.float32) * dinv


def _tc3_body(a0_ref, a1_ref, g_ref, dinv_ref, b_ref, batch_ref,
              wl_ref, bl_ref, out_ref, pooled_ref):
    i = pl.program_id(0)

    @pl.when(i == 0)
    def _init():
        pooled_ref[...] = jnp.zeros_like(pooled_ref)

    h = jax.nn.relu((a0_ref[...] + a1_ref[...] + g_ref[...]) * dinv_ref[...]
                    + b_ref[...])
    seg = lax.broadcasted_iota(jnp.int32, (G, BLK), 0)
    onehot = jnp.where(seg == batch_ref[0], 1.0, 0.0)
    pooled_ref[...] += jnp.dot(onehot, h, preferred_element_type=jnp.float32)

    @pl.when(i == NBLK - 1)
    def _fin():
        out_ref[...] = jnp.dot(pooled_ref[...], wl_ref[...],
                               preferred_element_type=jnp.float32) + bl_ref[...]


def _row_blk(d):
    return pl.BlockSpec((BLK, d), lambda i: (i, 0))


def _full(s0, s1):
    return pl.BlockSpec((s0, s1), lambda i: (0, 0))


def kernel(x, edge_index, batch, W1, b1, W2, b2, Wl, bl):
    src = edge_index[0].astype(jnp.int32)
    dst = edge_index[1].astype(jnp.int32)
    pad = (TOTC + max(KF - KS, 0)) * CH - E
    srcr = jnp.concatenate(
        [src, jnp.zeros((pad,), jnp.int32)]).reshape(-1, CH)
    dstr = jnp.concatenate(
        [dst, jnp.full((pad,), N, jnp.int32)]).reshape(-1, CH)
    idx_all = jnp.stack([srcr, dstr], axis=1)
    zeros2d = jnp.zeros((NPAD_S, D), jnp.float32)
    zeros1d = jnp.zeros((NPAD,), jnp.float32)
    ones_ch = jnp.ones((CH,), jnp.float32)

    deg_parts = _sc_degree(idx_all, ones_ch, zeros1d)
    m1 = pl.pallas_call(
        _tc0_body,
        grid=(NBLK,),
        in_specs=[_row_blk(D), _full(D, H)],
        out_specs=_row_blk(H),
        out_shape=jax.ShapeDtypeStruct((N, H), jnp.float32),
    )(x, W1)
    degt = deg_parts.reshape(NC, NPAD).T

    g1, dinv = pl.pallas_call(
        _tc1_body,
        grid=(NBLK,),
        in_specs=[_row_blk(H), _row_blk(2)],
        out_specs=[_row_blk(H), _row_blk(1)],
        out_shape=[jax.ShapeDtypeStruct((N, H), jnp.float32),
                   jax.ShapeDtypeStruct((N, 1), jnp.float32)],
    )(m1, degt)

    acc1 = _sc_scatter(g1, idx_all, zeros2d)

    g2 = pl.pallas_call(
        _tc2_body,
        grid=(NBLK,),
        in_specs=[_row_blk(H), _row_blk(H), _row_blk(H), _row_blk(1),
                  _full(1, H), _full(H, H)],
        out_specs=_row_blk(H),
        out_shape=jax.ShapeDtypeStruct((N, H), jnp.float32),
    )(acc1[0], acc1[1], g1, dinv, b1.reshape(1, H), W2)

    acc2 = _sc_scatter(g2, idx_all, zeros2d)

    out = pl.pallas_call(
        _tc3_body,
        grid=(NBLK,),
        in_specs=[_row_blk(H), _row_blk(H), _row_blk(H), _row_blk(1),
                  _full(1, H), pl.BlockSpec((1, 1, BLK), lambda i: (i, 0, 0)),
                  _full(H, O), _full(1, O)],
        out_specs=_full(G, O),
        out_shape=jax.ShapeDtypeStruct((G, O), jnp.float32),
        scratch_shapes=[pltpu.VMEM((G, H), jnp.float32)],
    )(acc2[0], acc2[1], g2, dinv, b2.reshape(1, H),
      batch.reshape(NBLK, 1, BLK).astype(jnp.int32), Wl, bl.reshape(1, O))

    return out


# speedup vs baseline: 1.0394x; 1.0394x over previous
import functools

import jax
import jax.numpy as jnp
from jax import lax
from jax.experimental import pallas as pl
from jax.experimental.pallas import tpu as pltpu
from jax.experimental.pallas import tpu_sc as plsc

N = 10000
E = 320000
D = 128
H = 128
O = 128
G = 64

NC = 2
NS = 16
NW = NC * NS
CH = 128
EPT = -(-E // NW)
K = -(-EPT // CH)
KF = 79
KS = 2 * K - KF
TOTC = NS * (KF + KS)
EPT_PAD = K * CH
NPAD = 10240
RPT = NPAD // NS
NPAD_S = 10112
RPT_S = NPAD_S // NS
BLK = 1000
NBLK = N // BLK

_sc_mesh = plsc.VectorSubcoreMesh(
    core_axis_name="c", subcore_axis_name="s", num_cores=NC, num_subcores=NS)


@functools.partial(
    pl.kernel,
    out_type=jax.ShapeDtypeStruct((NC, 1, NPAD), jnp.float32),
    mesh=_sc_mesh,
    scratch_types=[
        pltpu.VMEM((max(KF, KS), 2, CH), jnp.int32),
        pltpu.VMEM((CH,), jnp.float32),
        pltpu.VMEM_SHARED((NPAD,), jnp.float32),
    ],
)
def _sc_degree(idx_hbm, ones_hbm, zeros_hbm, out_hbm, idx_d, ones_v, deg_sh):
    cid = lax.axis_index("c")
    sid = lax.axis_index("s")
    base = jnp.where(cid == 0, sid * KF, NS * KF + sid * KS)
    kmine = jnp.where(cid == 0, KF, KS)
    r0 = sid * RPT
    pltpu.sync_copy(zeros_hbm.at[pl.ds(r0, RPT)], deg_sh.at[pl.ds(r0, RPT)])
    pltpu.sync_copy(idx_hbm.at[pl.ds(base, max(KF, KS))], idx_d)
    pltpu.sync_copy(ones_hbm, ones_v)
    plsc.subcore_barrier()

    @pl.loop(0, kmine)
    def _chunks(c):
        pltpu.sync_copy(ones_v, deg_sh.at[idx_d.at[c, 1]], add=True)

    plsc.subcore_barrier()
    pltpu.sync_copy(deg_sh.at[pl.ds(r0, RPT)],
                    out_hbm.at[cid, 0, pl.ds(r0, RPT)])


@functools.partial(
    pl.kernel,
    out_type=jax.ShapeDtypeStruct((NC, NPAD_S, D), jnp.float32),
    mesh=_sc_mesh,
    scratch_types=[
        pltpu.VMEM((4, 2, CH), jnp.int32),
        pltpu.VMEM((3, CH, D), jnp.float32),
        pltpu.VMEM_SHARED((NPAD_S, D), jnp.float32),
        pltpu.SemaphoreType.DMA,
        pltpu.SemaphoreType.DMA,
        pltpu.SemaphoreType.DMA,
    ],
)
def _sc_scatter(g_hbm, idx_hbm, zeros_hbm, out_hbm,
                idxb, rows, acc_sh, rsem, isem, ssem):
    cid = lax.axis_index("c")
    sid = lax.axis_index("s")
    base = jnp.where(cid == 0, sid * KF, NS * KF + sid * KS)
    kmine = jnp.where(cid == 0, KF, KS)
    r0 = sid * RPT_S
    pltpu.sync_copy(zeros_hbm.at[pl.ds(r0, RPT_S)], acc_sh.at[pl.ds(r0, RPT_S)])
    @pl.when(kmine > 0)
    def _prime():
        pltpu.sync_copy(idx_hbm.at[base], idxb.at[0])
        pltpu.async_copy(g_hbm.at[idxb.at[0, 0]], rows.at[0], rsem)
        pltpu.async_copy(idx_hbm.at[base + 1], idxb.at[1], isem)
        pltpu.async_copy(idx_hbm.at[base + 2], idxb.at[2], isem)
        pltpu.async_copy(idx_hbm.at[base + 3], idxb.at[3], isem)
        pltpu.make_async_copy(idx_hbm.at[0], idxb.at[1], isem).wait()
        pltpu.async_copy(g_hbm.at[idxb.at[1, 0]], rows.at[1], rsem)

    plsc.subcore_barrier()

    @pl.loop(0, kmine)
    def _chunks(c):
        buf = lax.rem(c, 3)
        ibuf = lax.rem(c, 4)
        pltpu.make_async_copy(
            zeros_hbm.at[pl.ds(0, CH)], rows.at[buf], rsem).wait()

        @pl.when(c >= 1)
        def _drain_scatter():
            pltpu.make_async_copy(
                zeros_hbm.at[pl.ds(0, CH)], rows.at[0], ssem).wait()

        @pl.when(c + 3 < kmine)
        def _next_idx():
            pltpu.async_copy(
                idx_hbm.at[base + c + 3], idxb.at[lax.rem(c + 3, 4)], isem)

        @pl.when(c + 2 < kmine)
        def _next_gather():
            i2 = lax.rem(c + 2, 4)
            pltpu.make_async_copy(idx_hbm.at[0], idxb.at[i2], isem).wait()
            pltpu.async_copy(
                g_hbm.at[idxb.at[i2, 0]], rows.at[lax.rem(c + 2, 3)], rsem)

        pltpu.async_copy(rows.at[buf], acc_sh.at[idxb.at[ibuf, 1]], ssem,
                         add=True)

    @pl.when(kmine > 0)
    def _tail():
        pltpu.make_async_copy(
            zeros_hbm.at[pl.ds(0, CH)], rows.at[0], ssem).wait()
        pltpu.make_async_copy(idx_hbm.at[0], idxb.at[0], isem).wait()

    plsc.subcore_barrier()
    pltpu.sync_copy(acc_sh.at[pl.ds(r0, RPT_S)],
                    out_hbm.at[cid, pl.ds(r0, RPT_S)])


def _tc1_body(x_ref, w1_ref, degt_ref, g1_ref, dinv_ref):
    deg = degt_ref[:, 0:1] + degt_ref[:, 1:2] + 1.0
    dinv = lax.rsqrt(deg)
    dinv_ref[...] = dinv
    g1_ref[...] = jnp.dot(x_ref[...], w1_ref[...],
                          preferred_element_type=jnp.float32) * dinv


def _tc2_body(a0_ref, a1_ref, g_ref, dinv_ref, b_ref, w_ref, out_ref):
    dinv = dinv_ref[...]
    h = jax.nn.relu((a0_ref[...] + a1_ref[...] + g_ref[...]) * dinv
                    + b_ref[...])
    out_ref[...] = jnp.dot(h, w_ref[...],
                           preferred_element_type=jnp.float32) * dinv


def _tc3_body(a0_ref, a1_ref, g_ref, dinv_ref, b_ref, batch_ref,
              wl_ref, bl_ref, out_ref, pooled_ref):
    i = pl.program_id(0)

    @pl.when(i == 0)
    def _init():
        pooled_ref[...] = jnp.zeros_like(pooled_ref)

    h = jax.nn.relu((a0_ref[...] + a1_ref[...] + g_ref[...]) * dinv_ref[...]
                    + b_ref[...])
    seg = lax.broadcasted_iota(jnp.int32, (G, BLK), 0)
    onehot = jnp.where(seg == batch_ref[0], 1.0, 0.0)
    pooled_ref[...] += jnp.dot(onehot, h, preferred_element_type=jnp.float32)

    @pl.when(i == NBLK - 1)
    def _fin():
        out_ref[...] = jnp.dot(pooled_ref[...], wl_ref[...],
                               preferred_element_type=jnp.float32) + bl_ref[...]


def _row_blk(d):
    return pl.BlockSpec((BLK, d), lambda i: (i, 0))


def _full(s0, s1):
    return pl.BlockSpec((s0, s1), lambda i: (0, 0))


def kernel(x, edge_index, batch, W1, b1, W2, b2, Wl, bl):
    src = edge_index[0].astype(jnp.int32)
    dst = edge_index[1].astype(jnp.int32)
    pad = (TOTC + max(KF - KS, 0)) * CH - E
    srcr = jnp.concatenate(
        [src, jnp.zeros((pad,), jnp.int32)]).reshape(-1, CH)
    dstr = jnp.concatenate(
        [dst, jnp.full((pad,), N, jnp.int32)]).reshape(-1, CH)
    idx_all = jnp.stack([srcr, dstr], axis=1)
    zeros2d = jnp.zeros((NPAD_S, D), jnp.float32)
    zeros1d = jnp.zeros((NPAD,), jnp.float32)
    ones_ch = jnp.ones((CH,), jnp.float32)

    deg_parts = _sc_degree(idx_all, ones_ch, zeros1d)
    degt = deg_parts.reshape(NC, NPAD).T

    g1, dinv = pl.pallas_call(
        _tc1_body,
        grid=(NBLK,),
        in_specs=[_row_blk(D), _full(D, H), _row_blk(2)],
        out_specs=[_row_blk(H), _row_blk(1)],
        out_shape=[jax.ShapeDtypeStruct((N, H), jnp.float32),
                   jax.ShapeDtypeStruct((N, 1), jnp.float32)],
    )(x, W1, degt)

    acc1 = _sc_scatter(g1, idx_all, zeros2d)

    g2 = pl.pallas_call(
        _tc2_body,
        grid=(NBLK,),
        in_specs=[_row_blk(H), _row_blk(H), _row_blk(H), _row_blk(1),
                  _full(1, H), _full(H, H)],
        out_specs=_row_blk(H),
        out_shape=jax.ShapeDtypeStruct((N, H), jnp.float32),
    )(acc1[0], acc1[1], g1, dinv, b1.reshape(1, H), W2)

    acc2 = _sc_scatter(g2, idx_all, zeros2d)

    out = pl.pallas_call(
        _tc3_body,
        grid=(NBLK,),
        in_specs=[_row_blk(H), _row_blk(H), _row_blk(H), _row_blk(1),
                  _full(1, H), pl.BlockSpec((1, 1, BLK), lambda i: (i, 0, 0)),
                  _full(H, O), _full(1, O)],
        out_specs=_full(G, O),
        out_shape=jax.ShapeDtypeStruct((G, O), jnp.float32),
        scratch_shapes=[pltpu.VMEM((G, H), jnp.float32)],
    )(acc2[0], acc2[1], g2, dinv, b2.reshape(1, H),
      batch.reshape(NBLK, 1, BLK).astype(jnp.int32), Wl, bl.reshape(1, O))

    return out
